# baseline (speedup 1.0000x reference)
import functools

import jax
import jax.numpy as jnp
from jax import lax
from jax.experimental import pallas as pl
from jax.experimental.pallas import tpu as pltpu
from jax.experimental.pallas import tpu_sc as plsc

_NUM_CORES = 2
_NUM_SUBCORES = 16
_NUM_TILES = _NUM_CORES * _NUM_SUBCORES
_CHUNK = 512


def kernel(word, table):
    bsz, seq = word.shape
    num_idx = bsz * seq
    dim = table.shape[1]
    n_per_tile = num_idx // _NUM_TILES
    n_chunks = n_per_tile // _CHUNK
    assert n_per_tile * _NUM_TILES == num_idx and n_chunks * _CHUNK == n_per_tile

    idx = word.reshape(num_idx).astype(jnp.int32)
    mesh = plsc.VectorSubcoreMesh(core_axis_name="c", subcore_axis_name="s")

    @functools.partial(
        pl.kernel,
        out_type=jax.ShapeDtypeStruct((num_idx, dim), table.dtype),
        mesh=mesh,
        scratch_types=[
            pltpu.VMEM((_CHUNK,), jnp.int32),
            pltpu.VMEM((_CHUNK, dim), table.dtype),
            pltpu.SemaphoreType.DMA,
        ],
        compiler_params=pltpu.CompilerParams(use_tc_tiling_on_sc=False),
    )
    def gather_kernel(tbl_hbm, idx_hbm, out_hbm, idx_v, rows_v, sem):
        wid = lax.axis_index("s") * _NUM_CORES + lax.axis_index("c")
        base = wid * n_per_tile

        @pl.loop(0, n_chunks)
        def _(c):
            off = base + c * _CHUNK
            pltpu.sync_copy(idx_hbm.at[pl.ds(off, _CHUNK)], idx_v)
            pltpu.async_copy(tbl_hbm.at[idx_v], rows_v, sem).wait()
            pltpu.sync_copy(rows_v, out_hbm.at[pl.ds(off, _CHUNK)])

    out = gather_kernel(table, idx)
    return out.reshape(bsz, seq, dim)


# speedup vs baseline: 1.0428x; 1.0428x over previous
import functools

import jax
import jax.numpy as jnp
from jax import lax
from jax.experimental import pallas as pl
from jax.experimental.pallas import tpu as pltpu
from jax.experimental.pallas import tpu_sc as plsc

_NUM_CORES = 2
_NUM_SUBCORES = 16
_NUM_TILES = _NUM_CORES * _NUM_SUBCORES
_CHUNK = 512


def kernel(word, table):
    bsz, seq = word.shape
    num_idx = bsz * seq
    dim = table.shape[1]
    n_per_tile = num_idx // _NUM_TILES
    n_chunks = n_per_tile // _CHUNK
    assert n_per_tile * _NUM_TILES == num_idx and n_chunks * _CHUNK == n_per_tile
    assert n_chunks >= 4 and n_chunks % 2 == 0

    idx = word.reshape(num_idx).astype(jnp.int32)
    mesh = plsc.VectorSubcoreMesh(core_axis_name="c", subcore_axis_name="s")

    @functools.partial(
        pl.kernel,
        out_type=jax.ShapeDtypeStruct((num_idx, dim), table.dtype),
        mesh=mesh,
        scratch_types=[
            pltpu.VMEM((n_per_tile,), jnp.int32),
            pltpu.VMEM((2, _CHUNK, dim), table.dtype),
            pltpu.SemaphoreType.DMA((2,)),
            pltpu.SemaphoreType.DMA((2,)),
        ],
        compiler_params=pltpu.CompilerParams(use_tc_tiling_on_sc=False),
    )
    def gather_kernel(tbl_hbm, idx_hbm, out_hbm, idx_v, rows_v, gat_sem, out_sem):
        wid = lax.axis_index("s") * _NUM_CORES + lax.axis_index("c")
        base = wid * n_per_tile
        pltpu.sync_copy(idx_hbm.at[pl.ds(base, n_per_tile)], idx_v)

        def start_gather(c, b):
            pltpu.async_copy(
                tbl_hbm.at[idx_v.at[pl.ds(c * _CHUNK, _CHUNK)]],
                rows_v.at[b],
                gat_sem.at[b],
            )

        def wait_gather(c, b):
            pltpu.make_async_copy(
                tbl_hbm.at[idx_v.at[pl.ds(c * _CHUNK, _CHUNK)]],
                rows_v.at[b],
                gat_sem.at[b],
            ).wait()

        def start_out(c, b):
            pltpu.async_copy(
                rows_v.at[b],
                out_hbm.at[pl.ds(base + c * _CHUNK, _CHUNK)],
                out_sem.at[b],
            )

        def wait_out(c, b):
            pltpu.make_async_copy(
                rows_v.at[b],
                out_hbm.at[pl.ds(base + c * _CHUNK, _CHUNK)],
                out_sem.at[b],
            ).wait()

        start_gather(0, 0)
        wait_gather(0, 0)
        start_out(0, 0)
        start_gather(1, 1)

        @pl.loop(0, (n_chunks - 2) // 2)
        def _(p):
            for k in range(2):
                c = 2 * p + 1 + k
                b = (1 + k) % 2
                nb = k % 2
                wait_gather(c, b)
                start_out(c, b)
                wait_out(c - 1, nb)
                start_gather(c + 1, nb)

        wait_gather(n_chunks - 1, 1)
        start_out(n_chunks - 1, 1)
        wait_out(n_chunks - 2, 0)
        wait_out(n_chunks - 1, 1)

    out = gather_kernel(table, idx)
    return out.reshape(bsz, seq, dim)


# speedup vs baseline: 1.0434x; 1.0007x over previous
import functools

import jax
import jax.numpy as jnp
from jax import lax
from jax.experimental import pallas as pl
from jax.experimental.pallas import tpu as pltpu
from jax.experimental.pallas import tpu_sc as plsc

_NUM_CORES = 2
_NUM_SUBCORES = 16
_NUM_TILES = _NUM_CORES * _NUM_SUBCORES
_CHUNK = 256
_NBUF = 4
_IF = 2


def kernel(word, table):
    bsz, seq = word.shape
    num_idx = bsz * seq
    dim = table.shape[1]
    n_per_tile = num_idx // _NUM_TILES
    n_chunks = n_per_tile // _CHUNK
    assert n_per_tile * _NUM_TILES == num_idx and n_chunks * _CHUNK == n_per_tile
    assert (n_chunks - 2 * _IF) % _NBUF == 0 and _NBUF >= 2 * _IF

    idx = word.reshape(num_idx).astype(jnp.int32)
    mesh = plsc.VectorSubcoreMesh(core_axis_name="c", subcore_axis_name="s")

    @functools.partial(
        pl.kernel,
        out_type=jax.ShapeDtypeStruct((num_idx, dim), table.dtype),
        mesh=mesh,
        scratch_types=[
            pltpu.VMEM((n_per_tile,), jnp.int32),
            pltpu.VMEM((_NBUF, _CHUNK, dim), table.dtype),
            pltpu.SemaphoreType.DMA((_NBUF,)),
            pltpu.SemaphoreType.DMA((_NBUF,)),
        ],
        compiler_params=pltpu.CompilerParams(use_tc_tiling_on_sc=False),
    )
    def gather_kernel(tbl_hbm, idx_hbm, out_hbm, idx_v, rows_v, gat_sem, out_sem):
        wid = lax.axis_index("s") * _NUM_CORES + lax.axis_index("c")
        base = wid * n_per_tile
        pltpu.sync_copy(idx_hbm.at[pl.ds(base, n_per_tile)], idx_v)

        def gather_copy(c, b):
            return pltpu.make_async_copy(
                tbl_hbm.at[idx_v.at[pl.ds(c * _CHUNK, _CHUNK)]],
                rows_v.at[b],
                gat_sem.at[b],
            )

        def out_copy(c, b):
            return pltpu.make_async_copy(
                rows_v.at[b],
                out_hbm.at[pl.ds(base + c * _CHUNK, _CHUNK)],
                out_sem.at[b],
            )

        def start_gather(c):
            gather_copy(c, c % _NBUF).start()

        def wait_gather(c):
            gather_copy(c, c % _NBUF).wait()

        def start_out(c):
            out_copy(c, c % _NBUF).start()

        def wait_out(c):
            out_copy(c, c % _NBUF).wait()

        for c in range(_IF):
            start_gather(c)
        for c in range(_IF):
            wait_gather(c)
            start_out(c)
            start_gather(c + _IF)

        @pl.loop(0, (n_chunks - 2 * _IF) // _NBUF)
        def _(p):
            for k in range(_NBUF):
                c = _IF + p * _NBUF + k
                wait_gather(c)
                start_out(c)
                wait_out(c + _IF - _NBUF)
                start_gather(c + _IF)

        for i in range(_IF):
            c = n_chunks - _IF + i
            wait_gather(c)
            start_out(c)
            wait_out(c + _IF - _NBUF)
        for i in range(_NBUF - _IF):
            wait_out(n_chunks - _NBUF + _IF + i)

    out = gather_kernel(table, idx)
    return out.reshape(bsz, seq, dim)


# speedup vs baseline: 1.0965x; 1.0508x over previous
import functools

import jax
import jax.numpy as jnp
from jax import lax
from jax.experimental import pallas as pl
from jax.experimental.pallas import tpu as pltpu
from jax.experimental.pallas import tpu_sc as plsc

_NUM_CORES = 2
_NUM_SUBCORES = 16
_NUM_TILES = _NUM_CORES * _NUM_SUBCORES
_CHUNK = 256
_NBUF = 4
_IF = 2


def kernel(word, table):
    bsz, seq = word.shape
    num_idx = bsz * seq
    dim = table.shape[1]
    n_per_tile = num_idx // _NUM_TILES
    n_chunks = n_per_tile // _CHUNK
    assert n_per_tile * _NUM_TILES == num_idx and n_chunks * _CHUNK == n_per_tile
    assert (n_chunks - 2 * _IF) % _NBUF == 0 and _NBUF >= 2 * _IF

    idx = word.reshape(num_idx).astype(jnp.int32)
    vocab = table.shape[0]
    tbl2 = jnp.pad(table, ((0, 0), (0, dim))).reshape(2 * vocab, dim)
    mesh = plsc.VectorSubcoreMesh(core_axis_name="c", subcore_axis_name="s")

    @functools.partial(
        pl.kernel,
        out_type=jax.ShapeDtypeStruct((num_idx, dim), table.dtype),
        mesh=mesh,
        scratch_types=[
            pltpu.VMEM((n_per_tile,), jnp.int32),
            pltpu.VMEM((_NBUF, _CHUNK, dim), table.dtype),
            pltpu.SemaphoreType.DMA((_NBUF,)),
            pltpu.SemaphoreType.DMA((_NBUF,)),
        ],
        compiler_params=pltpu.CompilerParams(use_tc_tiling_on_sc=False),
    )
    def gather_kernel(tbl_hbm, idx_hbm, out_hbm, idx_v, rows_v, gat_sem, out_sem):
        wid = lax.axis_index("s") * _NUM_CORES + lax.axis_index("c")
        base = wid * n_per_tile
        pltpu.sync_copy(idx_hbm.at[pl.ds(base, n_per_tile)], idx_v)

        @pl.loop(0, n_per_tile // 16)
        def _(i):
            sl = pl.ds(i * 16, 16)
            idx_v[sl] = idx_v[sl] * 2

        def gather_copy(c, b):
            return pltpu.make_async_copy(
                tbl_hbm.at[idx_v.at[pl.ds(c * _CHUNK, _CHUNK)]],
                rows_v.at[b],
                gat_sem.at[b],
            )

        def out_copy(c, b):
            return pltpu.make_async_copy(
                rows_v.at[b],
                out_hbm.at[pl.ds(base + c * _CHUNK, _CHUNK)],
                out_sem.at[b],
            )

        def start_gather(c):
            gather_copy(c, c % _NBUF).start()

        def wait_gather(c):
            gather_copy(c, c % _NBUF).wait()

        def start_out(c):
            out_copy(c, c % _NBUF).start()

        def wait_out(c):
            out_copy(c, c % _NBUF).wait()

        for c in range(_IF):
            start_gather(c)
        for c in range(_IF):
            wait_gather(c)
            start_out(c)
            start_gather(c + _IF)

        @pl.loop(0, (n_chunks - 2 * _IF) // _NBUF)
        def _(p):
            for k in range(_NBUF):
                c = _IF + p * _NBUF + k
                wait_gather(c)
                start_out(c)
                wait_out(c + _IF - _NBUF)
                start_gather(c + _IF)

        for i in range(_IF):
            c = n_chunks - _IF + i
            wait_gather(c)
            start_out(c)
            wait_out(c + _IF - _NBUF)
        for i in range(_NBUF - _IF):
            wait_out(n_chunks - _NBUF + _IF + i)

    out = gather_kernel(tbl2, idx)
    return out.reshape(bsz, seq, dim)
